# baseline (speedup 1.0000x reference)
import functools

import jax
import jax.numpy as jnp
from jax import lax
from jax.experimental import pallas as pl
from jax.experimental.pallas import tpu as pltpu
from jax.experimental.pallas import tpu_sc as plsc

_B = 16384
_S = 26
_DIM = 64
_TOTAL = _B * _S
_NW = 32
_PER_W = _TOTAL // _NW
_CHUNK = 128
_NCHUNK = _PER_W // _CHUNK


def _build():
    info = plsc.get_sparse_core_info()
    nc = info.num_cores
    mesh = plsc.VectorSubcoreMesh(core_axis_name="c", subcore_axis_name="s")

    @functools.partial(
        pl.kernel,
        mesh=mesh,
        out_type=jax.ShapeDtypeStruct((_S, _DIM, _B), jnp.float32),
        scratch_types=[
            pltpu.VMEM((_PER_W,), jnp.int32),
            pltpu.VMEM((_PER_W,), jnp.int32),
            pltpu.VMEM((_CHUNK, 128), jnp.float32),
            pltpu.VMEM((_DIM, _CHUNK), jnp.float32),
            pltpu.SemaphoreType.DMA,
        ],
        compiler_params=pltpu.CompilerParams(
            use_tc_tiling_on_sc=True, needs_layout_passes=False
        ),
    )
    def gather_kernel(idx_hbm, table_hbm, out_hbm, pairv, halfv, buf, tbuf, sem):
        wid = lax.axis_index("s") * nc + lax.axis_index("c")
        base = wid * _PER_W
        pltpu.sync_copy(idx_hbm.at[pl.ds(base, _PER_W)], pairv)

        @pl.loop(0, _PER_W, step=16)
        def _(t):
            v = pairv[pl.ds(t, 16)]
            halfv[pl.ds(t, 16)] = (v & 1) << 6
            pairv[pl.ds(t, 16)] = v >> 1

        lanes = [lax.iota(jnp.int32, 16) + 16 * g for g in range(8)]

        @pl.loop(0, _NCHUNK)
        def _(c):
            gchunk = wid * _NCHUNK + c
            j = gchunk >> 7
            i0 = pl.multiple_of((gchunk & 127) << 7, 128)
            pltpu.async_copy(
                table_hbm.at[pairv.at[pl.ds(c * _CHUNK, _CHUNK)]], buf, sem
            ).wait()
            for g in range(8):
                half_g = halfv[pl.ds(c * _CHUNK + 16 * g, 16)]

                @pl.loop(0, _DIM)
                def _(d, half_g=half_g, rows_g=lanes[g], g=g):
                    vals = plsc.load_gather(buf, [rows_g, half_g + d])
                    tbuf.at[d][pl.ds(16 * g, 16)] = vals

            for tr in range(8):
                pltpu.sync_copy(
                    tbuf.at[pl.ds(8 * tr, 8)],
                    out_hbm.at[j, pl.ds(8 * tr, 8), pl.ds(i0, _CHUNK)],
                )

    return gather_kernel


_gather = _build()


def kernel(indices, codewords):
    idx_flat = indices.T.reshape(-1).astype(jnp.int32)
    table2 = codewords.reshape(500000, 128)
    out = _gather(idx_flat, table2)
    return jnp.transpose(out, (2, 0, 1))


# speedup vs baseline: 1.1751x; 1.1751x over previous
import functools

import jax
import jax.numpy as jnp
from jax import lax
from jax.experimental import pallas as pl
from jax.experimental.pallas import tpu as pltpu
from jax.experimental.pallas import tpu_sc as plsc

_B = 16384
_S = 26
_DIM = 64
_TOTAL = _B * _S
_NW = 32
_PER_W = _TOTAL // _NW
_CHUNK = 128
_NCHUNK = _PER_W // _CHUNK


def _build():
    info = plsc.get_sparse_core_info()
    nc = info.num_cores
    mesh = plsc.VectorSubcoreMesh(core_axis_name="c", subcore_axis_name="s")

    @functools.partial(
        pl.kernel,
        mesh=mesh,
        out_type=jax.ShapeDtypeStruct((_S, _DIM, _B), jnp.float32),
        scratch_types=[
            pltpu.VMEM((_PER_W,), jnp.int32),
            pltpu.VMEM((_PER_W,), jnp.int32),
            pltpu.VMEM((2, _CHUNK, 128), jnp.float32),
            pltpu.VMEM((2, _DIM, _CHUNK), jnp.float32),
            pltpu.SemaphoreType.DMA((2,)),
            pltpu.SemaphoreType.DMA((2,)),
        ],
        compiler_params=pltpu.CompilerParams(
            use_tc_tiling_on_sc=True, needs_layout_passes=False
        ),
    )
    def gather_kernel(
        idx_hbm, table_hbm, out_hbm, pairv, halfv, buf, tbuf, gsem, osem
    ):
        wid = lax.axis_index("s") * nc + lax.axis_index("c")
        base = wid * _PER_W
        pltpu.sync_copy(idx_hbm.at[pl.ds(base, _PER_W)], pairv)

        @pl.loop(0, _PER_W, step=16)
        def _(t):
            v = pairv[pl.ds(t, 16)]
            halfv[pl.ds(t, 16)] = (v & 1) << 6
            pairv[pl.ds(t, 16)] = v >> 1

        lanes = [lax.iota(jnp.int32, 16) + 16 * g for g in range(8)]

        def gstart(b, c):
            pltpu.async_copy(
                table_hbm.at[pairv.at[pl.ds(c * _CHUNK, _CHUNK)]],
                buf.at[b],
                gsem.at[b],
            )

        def gwait(b):
            pltpu.make_async_copy(
                table_hbm.at[pairv.at[pl.ds(0, _CHUNK)]], buf.at[b], gsem.at[b]
            ).wait()

        def out_slice(c):
            gchunk = wid * _NCHUNK + c
            j = gchunk >> 7
            i0 = pl.multiple_of((gchunk & 127) << 7, 128)
            return out_hbm.at[j, :, pl.ds(i0, _CHUNK)]

        def ostart(b, c):
            pltpu.async_copy(tbuf.at[b], out_slice(c), osem.at[b])

        def owait(b):
            pltpu.make_async_copy(tbuf.at[b], out_slice(0), osem.at[b]).wait()

        def transpose(b, c):
            for g in range(8):
                half_g = halfv[pl.ds(c * _CHUNK + 16 * g, 16)]

                @pl.loop(0, _DIM, unroll=8)
                def _(d, half_g=half_g, rows_g=lanes[g], b=b, g=g):
                    vals = plsc.load_gather(buf.at[b], [rows_g, half_g + d])
                    tbuf.at[b].at[d][pl.ds(16 * g, 16)] = vals

        for b in range(2):
            gstart(b, b)
        for b in range(2):
            gwait(b)
            transpose(b, b)
            gstart(b, b + 2)
            ostart(b, b)

        @pl.loop(2, _NCHUNK - 2, step=2)
        def _(c):
            for b in range(2):
                gwait(b)
                owait(b)
                transpose(b, c + b)
                gstart(b, c + b + 2)
                ostart(b, c + b)

        for b in range(2):
            gwait(b)
            owait(b)
            transpose(b, _NCHUNK - 2 + b)
            ostart(b, _NCHUNK - 2 + b)
        for b in range(2):
            owait(b)

    return gather_kernel


_gather = _build()


def kernel(indices, codewords):
    idx_flat = indices.T.reshape(-1).astype(jnp.int32)
    table2 = codewords.reshape(500000, 128)
    out = _gather(idx_flat, table2)
    return jnp.transpose(out, (2, 0, 1))


# speedup vs baseline: 1.1796x; 1.0038x over previous
import functools

import jax
import jax.numpy as jnp
from jax import lax
from jax.experimental import pallas as pl
from jax.experimental.pallas import tpu as pltpu
from jax.experimental.pallas import tpu_sc as plsc

_B = 16384
_S = 26
_DIM = 64
_TOTAL = _B * _S
_NW = 32
_PER_W = _TOTAL // _NW
_CHUNK = 128
_NCHUNK = _PER_W // _CHUNK
_NBUF = 4


def _build():
    info = plsc.get_sparse_core_info()
    nc = info.num_cores
    mesh = plsc.VectorSubcoreMesh(core_axis_name="c", subcore_axis_name="s")

    @functools.partial(
        pl.kernel,
        mesh=mesh,
        out_type=jax.ShapeDtypeStruct((_S, 8, _B // _CHUNK, 8, _CHUNK), jnp.float32),
        scratch_types=[
            pltpu.VMEM((_PER_W,), jnp.int32),
            pltpu.VMEM((_NBUF, _CHUNK, _DIM), jnp.float32),
            pltpu.VMEM((2, 8, 8, _CHUNK), jnp.float32),
            pltpu.SemaphoreType.DMA((_NBUF,)),
            pltpu.SemaphoreType.DMA((2,)),
        ],
        compiler_params=pltpu.CompilerParams(
            use_tc_tiling_on_sc=False, needs_layout_passes=False
        ),
    )
    def gather_kernel(idx_hbm, table_hbm, out_hbm, idxv, buf, tbuf, gsem, osem):
        wid = lax.axis_index("s") * nc + lax.axis_index("c")
        base = wid * _PER_W
        pltpu.sync_copy(idx_hbm.at[pl.ds(base, _PER_W)], idxv)

        lanes = [lax.iota(jnp.int32, 16) + 16 * g for g in range(8)]

        def gstart(b, c):
            pltpu.async_copy(
                table_hbm.at[idxv.at[pl.ds(c * _CHUNK, _CHUNK)]],
                buf.at[b],
                gsem.at[b],
            )

        def gwait(b):
            pltpu.make_async_copy(
                table_hbm.at[idxv.at[pl.ds(0, _CHUNK)]], buf.at[b], gsem.at[b]
            ).wait()

        def out_slice(c):
            gchunk = wid * _NCHUNK + c
            j = gchunk >> 7
            tc = gchunk & 127
            return out_hbm.at[j, :, tc]

        def ostart(tb, c):
            pltpu.async_copy(tbuf.at[tb], out_slice(c), osem.at[tb])

        def owait(tb):
            pltpu.make_async_copy(tbuf.at[tb], out_slice(0), osem.at[tb]).wait()

        def transpose(b, tb):
            @pl.loop(0, _DIM, unroll=4)
            def _(d, b=b, tb=tb):
                tr = d >> 3
                s = d & 7
                colv = jnp.broadcast_to(d, (16,))
                for g in range(8):
                    vals = plsc.load_gather(buf.at[b], [lanes[g], colv])
                    tbuf.at[tb, tr, s][pl.ds(16 * g, 16)] = vals

        for b in range(_NBUF):
            gstart(b, b)
        for b in range(_NBUF):
            gwait(b)
            if b >= 2:
                owait(b & 1)
            transpose(b, b & 1)
            gstart(b, b + _NBUF)
            ostart(b & 1, b)

        @pl.loop(_NBUF, _NCHUNK - _NBUF, step=_NBUF)
        def _(c):
            for b in range(_NBUF):
                gwait(b)
                owait(b & 1)
                transpose(b, b & 1)
                gstart(b, c + b + _NBUF)
                ostart(b & 1, c + b)

        for b in range(_NBUF):
            gwait(b)
            owait(b & 1)
            transpose(b, b & 1)
            ostart(b & 1, _NCHUNK - _NBUF + b)
        for tb in range(2):
            owait(tb)

    return gather_kernel


_gather = _build()


def kernel(indices, codewords):
    idx_flat = indices.T.reshape(-1).astype(jnp.int32)
    out5 = _gather(idx_flat, codewords)
    return jnp.transpose(out5, (2, 4, 0, 1, 3)).reshape(_B, _S, _DIM)
